# baseline (speedup 1.0000x reference)
import functools

import jax
import jax.numpy as jnp
from jax import lax
from jax.experimental import pallas as pl
from jax.experimental.pallas import tpu as pltpu
from jax.experimental.pallas import tpu_sc as plsc

import numpy as np

_EPS = float(np.finfo(float).eps)

_NC = 2
_NS = 16
_L = 16
_NW = _NC * _NS

_CHUNK = 400


@functools.partial(jax.jit, static_argnums=(5, 6, 7))
def _sc_expand(xf, probs_f, mask_f, idx0_a, idx1_a, n_rows, coarse_num,
               event_num):
    kj = event_num // _L
    tail = event_num - _L
    rows_per_w = n_rows // _NW
    n_chunks = rows_per_w // _CHUNK
    tab = coarse_num * coarse_num

    mesh = plsc.VectorSubcoreMesh(core_axis_name="c", subcore_axis_name="s")

    @functools.partial(
        pl.kernel,
        out_type=jax.ShapeDtypeStruct((n_rows * event_num,), jnp.float32),
        mesh=mesh,
        compiler_params=pltpu.CompilerParams(needs_layout_passes=False,
                                             use_tc_tiling_on_sc=False),
        scratch_types=[
            pltpu.VMEM((event_num,), jnp.int32),
            pltpu.VMEM((event_num,), jnp.int32),
            pltpu.VMEM((tab,), jnp.float32),
            pltpu.VMEM((tab,), jnp.float32),
            pltpu.VMEM((_L,), jnp.float32),
            pltpu.VMEM((kj * _L,), jnp.int32),
            pltpu.VMEM((kj * _L,), jnp.float32),
            pltpu.VMEM((_L,), jnp.int32),
            pltpu.VMEM((_L,), jnp.float32),
            pltpu.VMEM((_CHUNK * coarse_num,), jnp.float32),
            pltpu.VMEM((_CHUNK * event_num,), jnp.float32),
        ],
    )
    def body(x_hbm, probs_hbm, mask_hbm, i0_hbm, i1_hbm, out_hbm,
             i0_v, i1_v, probs_v, mask_v, s_v, idx0_v, w_v, idxt_v, wt_v,
             cin_v, cout_v):
        cmax = coarse_num - 1

        pltpu.sync_copy(i0_hbm, i0_v)
        pltpu.sync_copy(i1_hbm, i1_v)
        pltpu.sync_copy(probs_hbm, probs_v)
        pltpu.sync_copy(mask_hbm, mask_v)

        cbase = jnp.minimum(lax.iota(jnp.int32, _L), cmax) * coarse_num
        s = jnp.zeros((_L,), jnp.float32)
        for i in range(coarse_num):
            fi = cbase + i
            p = plsc.load_gather(probs_v, [fi])
            m = plsc.load_gather(mask_v, [fi])
            s = s + (m * jnp.exp(-jnp.abs(p)) + _EPS)
        s_v[...] = s

        def weights_at(pos):
            i0 = jnp.clip(plsc.load_gather(i0_v, [pos]), 0, cmax)
            i1 = jnp.clip(plsc.load_gather(i1_v, [pos]), 0, cmax)
            fi = i0 * coarse_num + i1
            p = plsc.load_gather(probs_v, [fi])
            m = plsc.load_gather(mask_v, [fi])
            v = m * jnp.exp(-jnp.abs(p)) + _EPS
            sg = plsc.load_gather(s_v, [i0])
            return i0, v / sg

        for j in range(kj):
            i0, w = weights_at(lax.iota(jnp.int32, _L) + (_L * j))
            w_v[pl.ds(_L * j, _L)] = w
            idx0_v[pl.ds(_L * j, _L)] = i0
        i0t, wt = weights_at(lax.iota(jnp.int32, _L) + tail)
        wt_v[...] = wt
        idxt_v[...] = i0t

        wid = lax.axis_index("s") * _NC + lax.axis_index("c")
        row0 = wid * rows_per_w

        idx_regs = [idx0_v[pl.ds(_L * j, _L)] for j in range(kj)]
        w_regs = [w_v[pl.ds(_L * j, _L)] for j in range(kj)]
        idx_regs.append(idxt_v[...])
        w_regs.append(wt_v[...])
        offs = [_L * j for j in range(kj)] + [tail]

        def chunk_body(ci, _):
            base = row0 + ci * _CHUNK
            pltpu.sync_copy(
                x_hbm.at[pl.ds(base * coarse_num, _CHUNK * coarse_num)],
                cin_v)

            def row_body(r, _):
                rs = jnp.full((_L,), r * coarse_num, jnp.int32)
                rout = r * event_num
                for o, ir, wr in zip(offs, idx_regs, w_regs):
                    g = plsc.load_gather(cin_v, [rs + ir])
                    cout_v[pl.ds(rout + o, _L)] = g * wr
                return 0

            lax.fori_loop(0, _CHUNK, row_body, 0)
            pltpu.sync_copy(
                cout_v,
                out_hbm.at[pl.ds(base * event_num, _CHUNK * event_num)])
            return 0

        lax.fori_loop(0, n_chunks, chunk_body, 0)

    return body(xf, probs_f, mask_f, idx0_a, idx1_a)


def kernel(coarse_probs, probs, mask, indices):
    b, t, c = coarse_probs.shape
    k = indices.shape[0]
    xf = coarse_probs.reshape(b * t * c)
    idx = indices.astype(jnp.int32)
    out = _sc_expand(xf, probs.reshape(-1), mask.reshape(-1),
                     idx[:, 0], idx[:, 1], b * t, c, k)
    return out.reshape(b, t, k)


# speedup vs baseline: 1.5478x; 1.5478x over previous
import functools

import jax
import jax.numpy as jnp
from jax import lax
from jax.experimental import pallas as pl
from jax.experimental.pallas import tpu as pltpu
from jax.experimental.pallas import tpu_sc as plsc

import numpy as np

_EPS = float(np.finfo(float).eps)

_NC = 2
_NS = 16
_L = 16
_NW = _NC * _NS

_CHUNK = 400


@functools.partial(jax.jit, static_argnums=(5, 6, 7))
def _sc_expand(xf, probs_f, mask_f, idx0_a, idx1_a, n_rows, coarse_num,
               event_num):
    kj = event_num // _L
    tail = event_num - _L
    rows_per_w = n_rows // _NW
    n_chunks = rows_per_w // _CHUNK
    tab = coarse_num * coarse_num

    mesh = plsc.VectorSubcoreMesh(core_axis_name="c", subcore_axis_name="s")

    @functools.partial(
        pl.kernel,
        out_type=jax.ShapeDtypeStruct((n_rows * event_num,), jnp.float32),
        mesh=mesh,
        compiler_params=pltpu.CompilerParams(needs_layout_passes=False,
                                             use_tc_tiling_on_sc=False),
        scratch_types=[
            pltpu.VMEM((event_num,), jnp.int32),
            pltpu.VMEM((event_num,), jnp.int32),
            pltpu.VMEM((tab,), jnp.float32),
            pltpu.VMEM((tab,), jnp.float32),
            pltpu.VMEM((_L,), jnp.float32),
            pltpu.VMEM((kj * _L,), jnp.int32),
            pltpu.VMEM((kj * _L,), jnp.float32),
            pltpu.VMEM((_L,), jnp.int32),
            pltpu.VMEM((_L,), jnp.float32),
            pltpu.VMEM((_CHUNK * coarse_num,), jnp.float32),
            pltpu.VMEM((_CHUNK * coarse_num,), jnp.float32),
            pltpu.VMEM((_CHUNK * event_num,), jnp.float32),
            pltpu.VMEM((_CHUNK * event_num,), jnp.float32),
            pltpu.SemaphoreType.DMA,
            pltpu.SemaphoreType.DMA,
            pltpu.SemaphoreType.DMA,
            pltpu.SemaphoreType.DMA,
        ],
    )
    def body(x_hbm, probs_hbm, mask_hbm, i0_hbm, i1_hbm, out_hbm,
             i0_v, i1_v, probs_v, mask_v, s_v, idx0_v, w_v, idxt_v, wt_v,
             cin_a, cin_b, cout_a, cout_b, si_a, si_b, so_a, so_b):
        cmax = coarse_num - 1

        pltpu.sync_copy(i0_hbm, i0_v)
        pltpu.sync_copy(i1_hbm, i1_v)
        pltpu.sync_copy(probs_hbm, probs_v)
        pltpu.sync_copy(mask_hbm, mask_v)

        cbase = jnp.minimum(lax.iota(jnp.int32, _L), cmax) * coarse_num
        s = jnp.zeros((_L,), jnp.float32)
        for i in range(coarse_num):
            fi = cbase + i
            p = plsc.load_gather(probs_v, [fi])
            m = plsc.load_gather(mask_v, [fi])
            s = s + (m * jnp.exp(-jnp.abs(p)) + _EPS)
        s_v[...] = s

        def weights_at(pos):
            i0 = jnp.clip(plsc.load_gather(i0_v, [pos]), 0, cmax)
            i1 = jnp.clip(plsc.load_gather(i1_v, [pos]), 0, cmax)
            fi = i0 * coarse_num + i1
            p = plsc.load_gather(probs_v, [fi])
            m = plsc.load_gather(mask_v, [fi])
            v = m * jnp.exp(-jnp.abs(p)) + _EPS
            sg = plsc.load_gather(s_v, [i0])
            return i0, v / sg

        for j in range(kj):
            i0, w = weights_at(lax.iota(jnp.int32, _L) + (_L * j))
            w_v[pl.ds(_L * j, _L)] = w
            idx0_v[pl.ds(_L * j, _L)] = i0
        i0t, wt = weights_at(lax.iota(jnp.int32, _L) + tail)
        wt_v[...] = wt
        idxt_v[...] = i0t

        wid = lax.axis_index("s") * _NC + lax.axis_index("c")
        row0 = wid * rows_per_w

        idx_regs = [idx0_v[pl.ds(_L * j, _L)] for j in range(kj)]
        w_regs = [w_v[pl.ds(_L * j, _L)] for j in range(kj)]
        idx_regs.append(idxt_v[...])
        w_regs.append(wt_v[...])
        offs = [_L * j for j in range(kj)] + [tail]

        def in_copy(ci, buf, sem):
            base = row0 + ci * _CHUNK
            return pltpu.make_async_copy(
                x_hbm.at[pl.ds(base * coarse_num, _CHUNK * coarse_num)],
                buf, sem)

        def out_copy(ci, buf, sem):
            base = row0 + ci * _CHUNK
            return pltpu.make_async_copy(
                buf, out_hbm.at[pl.ds(base * event_num, _CHUNK * event_num)],
                sem)

        def compute(cin, cout):
            @plsc.parallel_loop(0, _CHUNK, unroll=4)
            def _(r):
                rs = jnp.full((_L,), r * coarse_num, jnp.int32)
                rout = r * event_num
                for o, ir, wr in zip(offs, idx_regs, w_regs):
                    g = plsc.load_gather(cin, [rs + ir])
                    cout[pl.ds(rout + o, _L)] = g * wr

        n_rounds = n_chunks // 2
        in_copy(0, cin_a, si_a).start()

        def round_body(di, _):
            ci0 = di * 2
            in_copy(ci0, cin_a, si_a).wait()
            in_copy(ci0 + 1, cin_b, si_b).start()

            @pl.when(di > 0)
            def _():
                out_copy(ci0, cout_a, so_a).wait()

            compute(cin_a, cout_a)
            out_copy(ci0, cout_a, so_a).start()

            in_copy(ci0 + 1, cin_b, si_b).wait()

            @pl.when(di < n_rounds - 1)
            def _():
                in_copy(ci0 + 2, cin_a, si_a).start()

            @pl.when(di > 0)
            def _():
                out_copy(ci0 + 1, cout_b, so_b).wait()

            compute(cin_b, cout_b)
            out_copy(ci0 + 1, cout_b, so_b).start()
            return 0

        lax.fori_loop(0, n_rounds, round_body, 0)
        out_copy(n_chunks - 2, cout_a, so_a).wait()
        out_copy(n_chunks - 1, cout_b, so_b).wait()

    return body(xf, probs_f, mask_f, idx0_a, idx1_a)


def kernel(coarse_probs, probs, mask, indices):
    b, t, c = coarse_probs.shape
    k = indices.shape[0]
    xf = coarse_probs.reshape(b * t * c)
    idx = indices.astype(jnp.int32)
    out = _sc_expand(xf, probs.reshape(-1), mask.reshape(-1),
                     idx[:, 0], idx[:, 1], b * t, c, k)
    return out.reshape(b, t, k)


# speedup vs baseline: 10.3981x; 6.7180x over previous
import functools

import jax
import jax.numpy as jnp
from jax import lax
from jax.experimental import pallas as pl
from jax.experimental.pallas import tpu as pltpu
from jax.experimental.pallas import tpu_sc as plsc

import numpy as np

_EPS = float(np.finfo(float).eps)

_NC = 2
_NS = 16
_L = 16
_NW = _NC * _NS

_TC = 8
_NRING = 4


@functools.partial(jax.jit, static_argnums=(5, 6, 7, 8, 9))
def _sc_expand(xt, probs_f, mask_f, idx0_a, idx1_a, bdim, tdim, coarse_num,
               maxkc, event_num):
    kj = -(-event_num // _L)
    tab = coarse_num * maxkc
    nt = tdim // _TC
    items = coarse_num * nt
    lanes_per_row = bdim

    mesh = plsc.VectorSubcoreMesh(core_axis_name="c", subcore_axis_name="s")

    @functools.partial(
        pl.kernel,
        out_type=jax.ShapeDtypeStruct((event_num, tdim, bdim), jnp.float32),
        mesh=mesh,
        compiler_params=pltpu.CompilerParams(needs_layout_passes=False,
                                             use_tc_tiling_on_sc=True),
        scratch_types=[
            pltpu.VMEM((event_num,), jnp.int32),
            pltpu.VMEM((event_num,), jnp.int32),
            pltpu.VMEM((tab,), jnp.float32),
            pltpu.VMEM((tab,), jnp.float32),
            pltpu.VMEM((_L,), jnp.float32),
            pltpu.VMEM((kj * _L,), jnp.float32),
            pltpu.VMEM((kj * _L,), jnp.int32),
            pltpu.VMEM((_TC, bdim), jnp.float32),
            pltpu.VMEM((_TC, bdim), jnp.float32),
            pltpu.VMEM((_TC, bdim), jnp.float32),
            pltpu.VMEM((_TC, bdim), jnp.float32),
            pltpu.VMEM((_TC, bdim), jnp.float32),
            pltpu.SemaphoreType.DMA,
            pltpu.SemaphoreType.DMA,
            pltpu.SemaphoreType.DMA,
            pltpu.SemaphoreType.DMA,
        ],
    )
    def body(x_hbm, probs_hbm, mask_hbm, i0_hbm, i1_hbm, out_hbm,
             i0_v, i1_v, probs_v, mask_v, s_v, w_v, inv_v,
             cin, co0, co1, co2, co3, so0, so1, so2, so3):
        cmax = coarse_num - 1
        kcmax = maxkc - 1
        couts = [co0, co1, co2, co3]
        sems = [so0, so1, so2, so3]

        pltpu.sync_copy(i0_hbm, i0_v)
        pltpu.sync_copy(i1_hbm, i1_v)
        pltpu.sync_copy(probs_hbm, probs_v)
        pltpu.sync_copy(mask_hbm, mask_v)

        cbase = jnp.minimum(lax.iota(jnp.int32, _L), cmax) * maxkc
        s = jnp.zeros((_L,), jnp.float32)
        for i in range(maxkc):
            fi = cbase + i
            p = plsc.load_gather(probs_v, [fi])
            m = plsc.load_gather(mask_v, [fi])
            s = s + (m * jnp.exp(-jnp.abs(p)) + _EPS)
        s_v[...] = s

        for j in range(kj):
            pos = lax.iota(jnp.int32, _L) + (_L * j)
            valid = pos < event_num
            posc = jnp.minimum(pos, event_num - 1)
            i0 = jnp.clip(plsc.load_gather(i0_v, [posc]), 0, cmax)
            i1 = jnp.clip(plsc.load_gather(i1_v, [posc]), 0, kcmax)
            fi = i0 * maxkc + i1
            p = plsc.load_gather(probs_v, [fi])
            m = plsc.load_gather(mask_v, [fi])
            v = m * jnp.exp(-jnp.abs(p)) + _EPS
            sg = plsc.load_gather(s_v, [i0])
            w_v[pl.ds(_L * j, _L)] = jnp.where(valid, v / sg, 0.0)
            plsc.store_scatter(inv_v, [fi], pos, mask=valid)

        wid = lax.axis_index("s") * _NC + lax.axis_index("c")
        n_items = (items - wid + _NW - 1) // _NW
        nvr = lanes_per_row // _L

        def item_body(n, _):
            item = wid + n * _NW
            c = item // nt
            t0 = (item % nt) * _TC
            pltpu.sync_copy(x_hbm.at[c, pl.ds(t0, _TC), :], cin)

            for i in range(maxkc):
                slot = i % _NRING
                kvec = plsc.load_gather(
                    inv_v, [jnp.full((_L,), c * maxkc + i, jnp.int32)])
                wvec = plsc.load_gather(w_v, [kvec])
                k_s = kvec[0]

                prev_exists = (i >= _NRING) | (n > 0)

                @pl.when(prev_exists)
                def _():
                    pltpu.make_async_copy(
                        couts[slot],
                        out_hbm.at[k_s, pl.ds(t0, _TC), :],
                        sems[slot]).wait()

                cout = couts[slot]

                @plsc.parallel_loop(0, nvr, unroll=4)
                def _(jv):
                    for srow in range(_TC):
                        cout[srow, pl.ds(jv * _L, _L)] = (
                            cin[srow, pl.ds(jv * _L, _L)] * wvec)

                pltpu.make_async_copy(
                    cout, out_hbm.at[k_s, pl.ds(t0, _TC), :],
                    sems[slot]).start()
            return 0

        lax.fori_loop(0, n_items, item_body, 0)

        for slot in range(_NRING):
            pltpu.make_async_copy(
                couts[slot], out_hbm.at[0, pl.ds(0, _TC), :],
                sems[slot]).wait()

    return body(xt, probs_f, mask_f, idx0_a, idx1_a)


def kernel(coarse_probs, probs, mask, indices):
    b, t, c = coarse_probs.shape
    k = indices.shape[0]
    kc = probs.shape[1]
    xt = jnp.transpose(coarse_probs, (2, 1, 0))
    idx = indices.astype(jnp.int32)
    out_t = _sc_expand(xt, probs.reshape(-1), mask.reshape(-1),
                       idx[:, 0], idx[:, 1], b, t, c, kc, k)
    return jnp.transpose(out_t, (2, 1, 0))
